# Initial kernel scaffold; baseline (speedup 1.0000x reference)
import jax
import jax.numpy as jnp
from jax.experimental import pallas as pl


def kernel(x_solvent, edge_index_solvent, edge_attr_solvent, mol_attr_solvent, x_solvent_batch, z_solute, pos_solute, solute_confs_batch, solute_mask, max_confs, gnn_Wm, gnn_bm, gnn_Wu, gnn_bu, emb, sch_fW1, sch_fb1, sch_fW2, sch_fb2, sch_W1, sch_W2, sch_b2, ffn_W1, ffn_b1, ffn_W2, ffn_b2):
    raise NotImplementedError("write your pallas kernel here")


# speedup vs baseline: 1.3840x; 1.3840x over previous
import functools

import jax
import jax.numpy as jnp
from jax import lax
from jax.experimental import pallas as pl
from jax.experimental.pallas import tpu as pltpu
from jax.experimental.pallas import tpu_sc as plsc

_NN = 10000
_NE = 320000
_D = 128
_DE = 16
_NM = 128
_A = 16
_NC = 1280
_NG = 50
_CUT = 10.0
_DEPTH = 3
_NBLK = 2

_SC_C = 2
_SC_S = 16
_NW = _SC_C * _SC_S
_EPW = _NE // _NW
_K = 80
_NCH = _EPW // _K
_IG = 25


def _ssp(x):
    return jnp.maximum(x, 0.0) + jnp.log1p(jnp.exp(-jnp.abs(x))) - 0.6931471805599453


def _eproj_body(ea_ref, w_ref, out_ref):
    out_ref[0] = jnp.dot(ea_ref[...], w_ref[0],
                         preferred_element_type=jnp.float32)


def _eproj(edge_attr, w_all):
    eb = 2000
    return pl.pallas_call(
        _eproj_body,
        grid=(_DEPTH, _NE // eb),
        in_specs=[
            pl.BlockSpec((eb, _DE), lambda l, e: (e, 0)),
            pl.BlockSpec((1, _DE, _D), lambda l, e: (l, 0, 0)),
        ],
        out_specs=pl.BlockSpec((1, eb, _D), lambda l, e: (l, e, 0)),
        out_shape=jax.ShapeDtypeStruct((_DEPTH, _NE, _D), jnp.float32),
    )(edge_attr, w_all)


def _lin_body(x_ref, w_ref, b_ref, out_ref):
    out_ref[...] = jnp.dot(x_ref[...], w_ref[...],
                           preferred_element_type=jnp.float32) + b_ref[...]


def _lin(x, w, b):
    rb = 1000
    return pl.pallas_call(
        _lin_body,
        grid=(_NN // rb,),
        in_specs=[
            pl.BlockSpec((rb, _D), lambda r: (r, 0)),
            pl.BlockSpec((_D, _D), lambda r: (0, 0)),
            pl.BlockSpec((1, _D), lambda r: (0, 0)),
        ],
        out_specs=pl.BlockSpec((rb, _D), lambda r: (r, 0)),
        out_shape=jax.ShapeDtypeStruct((_NN, _D), jnp.float32),
    )(x, w, b)


def _sc_agg_body(y_hbm, src_hbm, dst_hbm, ep_hbm, out_hbm,
                 sidx, didx, rows, erows, aggs, sem):
    cid = lax.axis_index("c")
    sid = lax.axis_index("s")
    wid = cid * _SC_S + sid

    def _zr(r, _):
        for j in range(_D // 16):
            rows[r, pl.ds(16 * j, 16)] = jnp.zeros((16,), jnp.float32)
        return _
    lax.fori_loop(0, _K, _zr, None)
    off = sid * 640
    nz = jnp.where(sid == _SC_S - 1, 5, 8)

    def _zs(zc, _):
        pltpu.sync_copy(rows, aggs.at[pl.ds(off + 80 * zc, 80)])
        return _
    lax.fori_loop(0, nz, _zs, None)
    plsc.subcore_barrier()

    def _group(g, _):
        pltpu.sync_copy(src_hbm.at[wid, g], sidx)
        pltpu.sync_copy(dst_hbm.at[wid, g], didx)

        def _chunk(c2, __):
            base = wid * _EPW + (g * _IG + c2) * _K
            pltpu.async_copy(y_hbm.at[sidx.at[c2]], rows, sem).wait()
            pltpu.sync_copy(ep_hbm.at[pl.ds(base, _K)], erows)

            def _row(r, ___):
                for j in range(_D // 16):
                    s = pl.ds(16 * j, 16)
                    rows[r, s] = jnp.maximum(rows[r, s] + erows[r, s], 0.0)
                return ___
            lax.fori_loop(0, _K, _row, None)
            pltpu.sync_copy(rows, aggs.at[didx.at[c2]], add=True)
            return __
        lax.fori_loop(0, _IG, _chunk, None)
        return _
    lax.fori_loop(0, _NCH // _IG, _group, None)
    plsc.subcore_barrier()

    def _out(zc, _):
        o = off + 80 * zc
        pltpu.sync_copy(aggs.at[pl.ds(o, 80)], out_hbm.at[cid, pl.ds(o, 80)])
        return _
    lax.fori_loop(0, nz, _out, None)


def _sc_agg(y, src3, dst3, ep_l):
    mesh = plsc.VectorSubcoreMesh(core_axis_name="c", subcore_axis_name="s")
    f = pl.kernel(
        _sc_agg_body,
        out_type=jax.ShapeDtypeStruct((_SC_C, _NN, _D), jnp.float32),
        mesh=mesh,
        scratch_types=[
            pltpu.VMEM((_IG, _K), jnp.int32),
            pltpu.VMEM((_IG, _K), jnp.int32),
            pltpu.VMEM((_K, _D), jnp.float32),
            pltpu.VMEM((_K, _D), jnp.float32),
            pltpu.VMEM_SHARED((_NN, _D), jnp.float32),
            pltpu.SemaphoreType.DMA,
        ],
    )
    return f(y, src3, dst3, ep_l)


def _upd_body(x_ref, a0_ref, a1_ref, wux_ref, wua_ref, bu_ref,
              wmx_ref, bm_ref, xn_ref, yn_ref):
    agg = a0_ref[0] + a1_ref[0]
    xn = jnp.maximum(
        jnp.dot(x_ref[...], wux_ref[...], preferred_element_type=jnp.float32)
        + jnp.dot(agg, wua_ref[...], preferred_element_type=jnp.float32)
        + bu_ref[...], 0.0)
    xn_ref[...] = xn
    yn_ref[...] = jnp.dot(xn, wmx_ref[...],
                          preferred_element_type=jnp.float32) + bm_ref[...]


def _upd(x, agg2, wux, wua, bu, wmx, bm):
    rb = 1000
    full = lambda r: (0, 0)
    return pl.pallas_call(
        _upd_body,
        grid=(_NN // rb,),
        in_specs=[
            pl.BlockSpec((rb, _D), lambda r: (r, 0)),
            pl.BlockSpec((1, rb, _D), lambda r: (0, r, 0)),
            pl.BlockSpec((1, rb, _D), lambda r: (1, r, 0)),
            pl.BlockSpec((_D, _D), full), pl.BlockSpec((_D, _D), full),
            pl.BlockSpec((1, _D), full),
            pl.BlockSpec((_D, _D), full), pl.BlockSpec((1, _D), full),
        ],
        out_specs=[
            pl.BlockSpec((rb, _D), lambda r: (r, 0)),
            pl.BlockSpec((rb, _D), lambda r: (r, 0)),
        ],
        out_shape=[
            jax.ShapeDtypeStruct((_NN, _D), jnp.float32),
            jax.ShapeDtypeStruct((_NN, _D), jnp.float32),
        ],
    )(x, agg2, agg2, wux, wua, bu, wmx, bm)


def _upd_body_a0(a0_ref, xn_ref):
    xn_ref[...] = a0_ref[...]


def _updh1_body(x_ref, a0_ref, a1_ref, wux_ref, wua_ref, bu_ref, bat_ref,
                sums_ref, cnt_ref):
    @pl.when(pl.program_id(0) == 0)
    def _init():
        sums_ref[...] = jnp.zeros_like(sums_ref)
        cnt_ref[...] = jnp.zeros_like(cnt_ref)

    agg = a0_ref[0] + a1_ref[0]
    xn = jnp.maximum(
        jnp.dot(x_ref[...], wux_ref[...], preferred_element_type=jnp.float32)
        + jnp.dot(agg, wua_ref[...], preferred_element_type=jnp.float32)
        + bu_ref[...], 0.0)
    b = bat_ref[0, 0, :]
    rows = lax.broadcasted_iota(jnp.int32, (_NM, b.shape[0]), 0)
    oh = (rows == b[None, :]).astype(jnp.float32)
    sums_ref[...] += jnp.dot(oh, xn, preferred_element_type=jnp.float32)
    cnt_ref[...] += jnp.sum(oh, axis=1, keepdims=True)


def _updh1(x, agg2, wux, wua, bu, bat3):
    rb = 1000
    full = lambda r: (0, 0)
    return pl.pallas_call(
        _updh1_body,
        grid=(_NN // rb,),
        in_specs=[
            pl.BlockSpec((rb, _D), lambda r: (r, 0)),
            pl.BlockSpec((1, rb, _D), lambda r: (0, r, 0)),
            pl.BlockSpec((1, rb, _D), lambda r: (1, r, 0)),
            pl.BlockSpec((_D, _D), full), pl.BlockSpec((_D, _D), full),
            pl.BlockSpec((1, _D), full),
            pl.BlockSpec((1, 1, rb), lambda r: (r, 0, 0)),
        ],
        out_specs=[
            pl.BlockSpec((_NM, _D), full),
            pl.BlockSpec((_NM, 1), full),
        ],
        out_shape=[
            jax.ShapeDtypeStruct((_NM, _D), jnp.float32),
            jax.ShapeDtypeStruct((_NM, 1), jnp.float32),
        ],
    )(x, agg2, agg2, wux, wua, bu, bat3)


_CB = 8
_CA = _CB * _A


def _schnet_body(pxi_ref, pxj_ref, pyi_ref, pyj_ref, pzi_ref, pzj_ref,
                 z_ref, emb_ref,
                 fw1a_ref, fb1a_ref, fw2a_ref, fb2a_ref,
                 w1a_ref, w2a_ref, b2a_ref,
                 fw1b_ref, fb1b_ref, fw2b_ref, fb2b_ref,
                 w1b_ref, w2b_ref, b2b_ref,
                 h2_ref):
    dx = pxi_ref[...] - pxj_ref[...]
    dy = pyi_ref[...] - pyj_ref[...]
    dz = pzi_ref[...] - pzj_ref[...]
    d = jnp.sqrt(dx * dx + dy * dy + dz * dz + 1e-12)

    mu = lax.broadcasted_iota(jnp.int32, (1, _NG), 1).astype(
        jnp.float32) * (_CUT / (_NG - 1))
    rbf2 = jnp.exp(-10.0 * (d - mu) ** 2)

    cc = 0.5 * (jnp.cos(jnp.pi * d / _CUT) + 1.0) * (d < _CUT)
    r = lax.broadcasted_iota(jnp.int32, (_CA * _A, 1), 0)
    gate = cc * ((r // _A) % _A != r % _A)

    zi = z_ref[...]
    ks = lax.broadcasted_iota(jnp.int32, (_CA, 100), 1)
    oh = (zi == ks).astype(jnp.float32)
    h = jnp.dot(oh, emb_ref[...], preferred_element_type=jnp.float32)

    blocks = [
        (fw1a_ref, fb1a_ref, fw2a_ref, fb2a_ref, w1a_ref, w2a_ref, b2a_ref),
        (fw1b_ref, fb1b_ref, fw2b_ref, fb2b_ref, w1b_ref, w2b_ref, b2b_ref),
    ]
    for fw1, fb1, fw2, fb2, w1, w2, b2 in blocks:
        t1 = _ssp(jnp.dot(rbf2, fw1[...],
                          preferred_element_type=jnp.float32) + fb1[...])
        filt = (jnp.dot(t1, fw2[...],
                        preferred_element_type=jnp.float32) + fb2[...]) * gate
        v = jnp.dot(h, w1[...], preferred_element_type=jnp.float32)
        f4 = filt.reshape(_CB, _A, _A, _D)
        v4 = v.reshape(_CB, 1, _A, _D)
        m = jnp.sum(f4 * v4, axis=2).reshape(_CA, _D)
        h = h + jnp.dot(_ssp(m), w2[...],
                        preferred_element_type=jnp.float32) + b2[...]

    h2_ref[...] = jnp.sum(h.reshape(_CB, _A, _D), axis=1)


def _schnet(pairs, zc, emb,
            sch_fW1, sch_fb1, sch_fW2, sch_fb2, sch_W1, sch_W2, sch_b2):
    full = lambda c: (0, 0)
    pb = lambda c: (c, 0)
    wspecs = []
    wvals = []
    for k in range(_NBLK):
        wspecs += [
            pl.BlockSpec((_NG, _D), full), pl.BlockSpec((1, _D), full),
            pl.BlockSpec((_D, _D), full), pl.BlockSpec((1, _D), full),
            pl.BlockSpec((_D, _D), full), pl.BlockSpec((_D, _D), full),
            pl.BlockSpec((1, _D), full),
        ]
        wvals += [sch_fW1[k], sch_fb1[k].reshape(1, _D),
                  sch_fW2[k], sch_fb2[k].reshape(1, _D),
                  sch_W1[k], sch_W2[k], sch_b2[k].reshape(1, _D)]
    return pl.pallas_call(
        _schnet_body,
        grid=(_NC // _CB,),
        in_specs=[pl.BlockSpec((_CA * _A, 1), pb)] * 6 + [
            pl.BlockSpec((_CA, 1), pb),
            pl.BlockSpec((100, _D), full),
        ] + wspecs,
        out_specs=pl.BlockSpec((_CB, _D), pb),
        out_shape=jax.ShapeDtypeStruct((_NC, _D), jnp.float32),
    )(*pairs, zc, emb, *wvals)


def _final_body(sums_ref, cnt_ref, h2_ref, mask_ref,
                w1a_ref, w1b_ref, b1_ref, w2_ref, b2_ref, out_ref):
    h1 = sums_ref[...] / jnp.maximum(cnt_ref[...], 1.0)
    g1 = jnp.dot(h1, w1a_ref[...], preferred_element_type=jnp.float32)
    cc = lax.broadcasted_iota(jnp.int32, (_NC, _NM), 0) // 10
    mm = lax.broadcasted_iota(jnp.int32, (_NC, _NM), 1)
    rmat = (cc == mm).astype(jnp.float32)
    g1r = jnp.dot(rmat, g1, preferred_element_type=jnp.float32)
    h2m = h2_ref[...] * mask_ref[...]
    t = jnp.maximum(
        jnp.dot(h2m, w1b_ref[...], preferred_element_type=jnp.float32)
        + g1r + b1_ref[...], 0.0)
    out_ref[...] = (jnp.dot(t, w2_ref[...],
                            preferred_element_type=jnp.float32)
                    + b2_ref[...]) * mask_ref[...]


def _final(sums, cnt, h2, maskf, w1a, w1b, b1, w2, b2):
    full = lambda: (0, 0)
    return pl.pallas_call(
        _final_body,
        in_specs=[
            pl.BlockSpec(sums.shape, None), pl.BlockSpec(cnt.shape, None),
            pl.BlockSpec(h2.shape, None), pl.BlockSpec(maskf.shape, None),
            pl.BlockSpec(w1a.shape, None), pl.BlockSpec(w1b.shape, None),
            pl.BlockSpec(b1.shape, None), pl.BlockSpec(w2.shape, None),
            pl.BlockSpec(b2.shape, None),
        ],
        out_specs=pl.BlockSpec((_NC, 1), None),
        out_shape=jax.ShapeDtypeStruct((_NC, 1), jnp.float32),
    )(sums, cnt, h2, maskf, w1a, w1b, b1, w2, b2)


def kernel(x_solvent, edge_index_solvent, edge_attr_solvent, mol_attr_solvent,
           x_solvent_batch, z_solute, pos_solute, solute_confs_batch,
           solute_mask, max_confs,
           gnn_Wm, gnn_bm, gnn_Wu, gnn_bu, emb,
           sch_fW1, sch_fb1, sch_fW2, sch_fb2, sch_W1, sch_W2, sch_b2,
           ffn_W1, ffn_b1, ffn_W2, ffn_b2):
    src3 = edge_index_solvent[0].astype(jnp.int32).reshape(
        _NW, _NCH // _IG, _IG, _K)
    dst3 = edge_index_solvent[1].astype(jnp.int32).reshape(
        _NW, _NCH // _IG, _IG, _K)
    wme = jnp.stack([w[_D:, :] for w in gnn_Wm])
    wmx = [w[:_D, :] for w in gnn_Wm]
    wux = [w[:_D, :] for w in gnn_Wu]
    wua = [w[_D:, :] for w in gnn_Wu]
    bm2 = [b.reshape(1, _D) for b in gnn_bm]
    bu2 = [b.reshape(1, _D) for b in gnn_bu]
    bat3 = x_solvent_batch.astype(jnp.int32).reshape(10, 1, 1000)

    npair = _NC * _A * _A

    def _pi(col):
        return jnp.broadcast_to(col[:, None], (_NC * _A, _A)).reshape(npair, 1)

    def _pj(col):
        return jnp.broadcast_to(col.reshape(_NC, 1, _A),
                                (_NC, _A, _A)).reshape(npair, 1)

    pairs = []
    for c in range(3):
        col = pos_solute[:, c]
        pairs += [_pi(col), _pj(col)]
    zc = z_solute.astype(jnp.int32).reshape(_NC * _A, 1)
    maskf = solute_mask.astype(jnp.float32).reshape(_NC, 1)

    ep = _eproj(edge_attr_solvent, wme)
    x = x_solvent
    y = _lin(x, wmx[0], bm2[0])
    for l in range(_DEPTH):
        agg2 = _sc_agg(y, src3, dst3, ep[l])
        if l < _DEPTH - 1:
            x, y = _upd(x, agg2, wux[l], wua[l], bu2[l],
                        wmx[l + 1], bm2[l + 1])
        else:
            sums, cnt = _updh1(x, agg2, wux[l], wua[l], bu2[l], bat3)

    h2 = _schnet(pairs, zc, emb,
                 sch_fW1, sch_fb1, sch_fW2, sch_fb2, sch_W1, sch_W2, sch_b2)

    out = _final(sums, cnt, h2, maskf,
                 ffn_W1[:_D, :], ffn_W1[_D:, :], ffn_b1.reshape(1, 2 * _D),
                 ffn_W2, ffn_b2.reshape(1, 1))
    return out[:, 0]


# speedup vs baseline: 2.0851x; 1.5066x over previous
import functools

import jax
import jax.numpy as jnp
from jax import lax
from jax.experimental import pallas as pl
from jax.experimental.pallas import tpu as pltpu
from jax.experimental.pallas import tpu_sc as plsc

_NN = 10000
_NE = 320000
_D = 128
_DE = 16
_NM = 128
_A = 16
_NC = 1280
_NG = 50
_CUT = 10.0
_DEPTH = 3
_NBLK = 2

_SC_C = 2
_SC_S = 16
_NW = _SC_C * _SC_S
_EPW = _NE // _NW
_K = 80
_NCH = _EPW // _K
_IG = 25


def _ssp(x):
    return jnp.maximum(x, 0.0) + jnp.log1p(jnp.exp(-jnp.abs(x))) - 0.6931471805599453


def _eproj_body(ea_ref, w_ref, out_ref):
    out_ref[0] = jnp.dot(ea_ref[...], w_ref[0],
                         preferred_element_type=jnp.float32)


def _eproj(edge_attr, w_all):
    eb = 2000
    return pl.pallas_call(
        _eproj_body,
        grid=(_DEPTH, _NE // eb),
        in_specs=[
            pl.BlockSpec((eb, _DE), lambda l, e: (e, 0)),
            pl.BlockSpec((1, _DE, _D), lambda l, e: (l, 0, 0)),
        ],
        out_specs=pl.BlockSpec((1, eb, _D), lambda l, e: (l, e, 0)),
        out_shape=jax.ShapeDtypeStruct((_DEPTH, _NE, _D), jnp.float32),
    )(edge_attr, w_all)


def _lin_body(x_ref, w_ref, b_ref, out_ref):
    out_ref[...] = jnp.dot(x_ref[...], w_ref[...],
                           preferred_element_type=jnp.float32) + b_ref[...]


def _lin(x, w, b):
    rb = 1000
    return pl.pallas_call(
        _lin_body,
        grid=(_NN // rb,),
        in_specs=[
            pl.BlockSpec((rb, _D), lambda r: (r, 0)),
            pl.BlockSpec((_D, _D), lambda r: (0, 0)),
            pl.BlockSpec((1, _D), lambda r: (0, 0)),
        ],
        out_specs=pl.BlockSpec((rb, _D), lambda r: (r, 0)),
        out_shape=jax.ShapeDtypeStruct((_NN, _D), jnp.float32),
    )(x, w, b)


def _sc_agg_body(y_hbm, src_hbm, dst_hbm, ep_hbm, out_hbm,
                 sidx, didx, rows, erows, aggs, sem):
    cid = lax.axis_index("c")
    sid = lax.axis_index("s")
    wid = cid * _SC_S + sid

    def _zr(r, _):
        for j in range(_D // 16):
            rows[r, pl.ds(16 * j, 16)] = jnp.zeros((16,), jnp.float32)
        return _
    lax.fori_loop(0, _K, _zr, None)
    off = sid * 640
    nz = jnp.where(sid == _SC_S - 1, 5, 8)

    def _zs(zc, _):
        pltpu.sync_copy(rows, aggs.at[pl.ds(off + 80 * zc, 80)])
        return _
    lax.fori_loop(0, nz, _zs, None)
    plsc.subcore_barrier()

    def _group(g, _):
        pltpu.sync_copy(src_hbm.at[wid, g], sidx)
        pltpu.sync_copy(dst_hbm.at[wid, g], didx)

        def _chunk(c2, __):
            base = wid * _EPW + (g * _IG + c2) * _K
            pltpu.async_copy(y_hbm.at[sidx.at[c2]], rows, sem).wait()
            pltpu.sync_copy(ep_hbm.at[pl.ds(base, _K)], erows)

            def _row(r, ___):
                for j in range(_D // 16):
                    s = pl.ds(16 * j, 16)
                    rows[r, s] = jnp.maximum(rows[r, s] + erows[r, s], 0.0)
                return ___
            lax.fori_loop(0, _K, _row, None)
            pltpu.sync_copy(rows, aggs.at[didx.at[c2]], add=True)
            return __
        lax.fori_loop(0, _IG, _chunk, None)
        return _
    lax.fori_loop(0, _NCH // _IG, _group, None)
    plsc.subcore_barrier()

    def _out(zc, _):
        o = off + 80 * zc
        pltpu.sync_copy(aggs.at[pl.ds(o, 80)], out_hbm.at[cid, pl.ds(o, 80)])
        return _
    lax.fori_loop(0, nz, _out, None)


def _sc_agg(y, src3, dst3, ep_l):
    mesh = plsc.VectorSubcoreMesh(core_axis_name="c", subcore_axis_name="s")
    f = pl.kernel(
        _sc_agg_body,
        out_type=jax.ShapeDtypeStruct((_SC_C, _NN, _D), jnp.float32),
        mesh=mesh,
        scratch_types=[
            pltpu.VMEM((_IG, _K), jnp.int32),
            pltpu.VMEM((_IG, _K), jnp.int32),
            pltpu.VMEM((_K, _D), jnp.float32),
            pltpu.VMEM((_K, _D), jnp.float32),
            pltpu.VMEM_SHARED((_NN, _D), jnp.float32),
            pltpu.SemaphoreType.DMA,
        ],
    )
    return f(y, src3, dst3, ep_l)


def _upd_body(x_ref, a0_ref, a1_ref, wux_ref, wua_ref, bu_ref,
              wmx_ref, bm_ref, xn_ref, yn_ref):
    agg = a0_ref[0] + a1_ref[0]
    xn = jnp.maximum(
        jnp.dot(x_ref[...], wux_ref[...], preferred_element_type=jnp.float32)
        + jnp.dot(agg, wua_ref[...], preferred_element_type=jnp.float32)
        + bu_ref[...], 0.0)
    xn_ref[...] = xn
    yn_ref[...] = jnp.dot(xn, wmx_ref[...],
                          preferred_element_type=jnp.float32) + bm_ref[...]


def _upd(x, agg2, wux, wua, bu, wmx, bm):
    rb = 1000
    full = lambda r: (0, 0)
    return pl.pallas_call(
        _upd_body,
        grid=(_NN // rb,),
        in_specs=[
            pl.BlockSpec((rb, _D), lambda r: (r, 0)),
            pl.BlockSpec((1, rb, _D), lambda r: (0, r, 0)),
            pl.BlockSpec((1, rb, _D), lambda r: (1, r, 0)),
            pl.BlockSpec((_D, _D), full), pl.BlockSpec((_D, _D), full),
            pl.BlockSpec((1, _D), full),
            pl.BlockSpec((_D, _D), full), pl.BlockSpec((1, _D), full),
        ],
        out_specs=[
            pl.BlockSpec((rb, _D), lambda r: (r, 0)),
            pl.BlockSpec((rb, _D), lambda r: (r, 0)),
        ],
        out_shape=[
            jax.ShapeDtypeStruct((_NN, _D), jnp.float32),
            jax.ShapeDtypeStruct((_NN, _D), jnp.float32),
        ],
    )(x, agg2, agg2, wux, wua, bu, wmx, bm)


def _upd_body_a0(a0_ref, xn_ref):
    xn_ref[...] = a0_ref[...]


def _updh1_body(x_ref, a0_ref, a1_ref, wux_ref, wua_ref, bu_ref, bat_ref,
                sums_ref, cnt_ref):
    @pl.when(pl.program_id(0) == 0)
    def _init():
        sums_ref[...] = jnp.zeros_like(sums_ref)
        cnt_ref[...] = jnp.zeros_like(cnt_ref)

    agg = a0_ref[0] + a1_ref[0]
    xn = jnp.maximum(
        jnp.dot(x_ref[...], wux_ref[...], preferred_element_type=jnp.float32)
        + jnp.dot(agg, wua_ref[...], preferred_element_type=jnp.float32)
        + bu_ref[...], 0.0)
    b = bat_ref[0, 0, :]
    rows = lax.broadcasted_iota(jnp.int32, (_NM, b.shape[0]), 0)
    oh = (rows == b[None, :]).astype(jnp.float32)
    sums_ref[...] += jnp.dot(oh, xn, preferred_element_type=jnp.float32)
    cnt_ref[...] += jnp.sum(oh, axis=1, keepdims=True)


def _updh1(x, agg2, wux, wua, bu, bat3):
    rb = 1000
    full = lambda r: (0, 0)
    return pl.pallas_call(
        _updh1_body,
        grid=(_NN // rb,),
        in_specs=[
            pl.BlockSpec((rb, _D), lambda r: (r, 0)),
            pl.BlockSpec((1, rb, _D), lambda r: (0, r, 0)),
            pl.BlockSpec((1, rb, _D), lambda r: (1, r, 0)),
            pl.BlockSpec((_D, _D), full), pl.BlockSpec((_D, _D), full),
            pl.BlockSpec((1, _D), full),
            pl.BlockSpec((1, 1, rb), lambda r: (r, 0, 0)),
        ],
        out_specs=[
            pl.BlockSpec((_NM, _D), full),
            pl.BlockSpec((_NM, 1), full),
        ],
        out_shape=[
            jax.ShapeDtypeStruct((_NM, _D), jnp.float32),
            jax.ShapeDtypeStruct((_NM, 1), jnp.float32),
        ],
    )(x, agg2, agg2, wux, wua, bu, bat3)


_CB = 8
_CA = _CB * _A


_NP = _CA * _A


def _schnet_body(pxi_ref, pxj_ref, pyi_ref, pyj_ref, pzi_ref, pzj_ref,
                 z_ref, embt_ref,
                 fw1a_ref, fb1a_ref, fw2a_ref, fb2a_ref,
                 w1a_ref, w2a_ref, b2a_ref,
                 fw1b_ref, fb1b_ref, fw2b_ref, fb2b_ref,
                 w1b_ref, w2b_ref, b2b_ref,
                 h2t_ref):
    dx = pxi_ref[0] - pxj_ref[0]
    dy = pyi_ref[0] - pyj_ref[0]
    dz = pzi_ref[0] - pzj_ref[0]
    d = jnp.sqrt(dx * dx + dy * dy + dz * dz + 1e-12)

    mu = lax.broadcasted_iota(jnp.int32, (_NG, 1), 0).astype(
        jnp.float32) * (_CUT / (_NG - 1))
    rbft = jnp.exp(-10.0 * (d - mu) ** 2)

    p = lax.broadcasted_iota(jnp.int32, (1, _NP), 1)
    gate = (0.5 * (jnp.cos(jnp.pi * d / _CUT) + 1.0) * (d < _CUT)
            * ((p // _A) % _A != p % _A))

    rows_a = lax.broadcasted_iota(jnp.int32, (_CA, _NP), 0)
    cols_p = lax.broadcasted_iota(jnp.int32, (_CA, _NP), 1)
    bsel = ((rows_a // _A == cols_p // (_A * _A))
            & (rows_a % _A == cols_p % _A)).astype(jnp.float32)
    s2 = (lax.broadcasted_iota(jnp.int32, (_NP, _CA), 0) // _A
          == lax.broadcasted_iota(jnp.int32, (_NP, _CA), 1)
          ).astype(jnp.float32)
    s3 = (lax.broadcasted_iota(jnp.int32, (_CA, _CB), 0) // _A
          == lax.broadcasted_iota(jnp.int32, (_CA, _CB), 1)
          ).astype(jnp.float32)

    za = z_ref[0]
    ks = lax.broadcasted_iota(jnp.int32, (100, _CA), 0)
    oht = (za == ks).astype(jnp.float32)
    ht = jnp.dot(embt_ref[...], oht, preferred_element_type=jnp.float32)

    blocks = [
        (fw1a_ref, fb1a_ref, fw2a_ref, fb2a_ref, w1a_ref, w2a_ref, b2a_ref),
        (fw1b_ref, fb1b_ref, fw2b_ref, fb2b_ref, w1b_ref, w2b_ref, b2b_ref),
    ]
    for fw1t, fb1c, fw2t, fb2c, w1t, w2t, b2c in blocks:
        t1 = _ssp(jnp.dot(fw1t[...], rbft,
                          preferred_element_type=jnp.float32) + fb1c[...])
        filt = (jnp.dot(fw2t[...], t1,
                        preferred_element_type=jnp.float32) + fb2c[...]) * gate
        vt = jnp.dot(w1t[...], ht, preferred_element_type=jnp.float32)
        vp = jnp.dot(vt, bsel, preferred_element_type=jnp.float32)
        mt = jnp.dot(filt * vp, s2, preferred_element_type=jnp.float32)
        ht = ht + jnp.dot(w2t[...], _ssp(mt),
                          preferred_element_type=jnp.float32) + b2c[...]

    h2t_ref[0] = jnp.dot(ht, s3, preferred_element_type=jnp.float32)


def _schnet(pairs, zrow, embt,
            sch_fW1, sch_fb1, sch_fW2, sch_fb2, sch_W1, sch_W2, sch_b2):
    full = lambda c: (0, 0)
    rowb = lambda c: (c, 0)
    colb = lambda c: (0, c)
    wspecs = []
    wvals = []
    for k in range(_NBLK):
        wspecs += [
            pl.BlockSpec((_D, _NG), full), pl.BlockSpec((_D, 1), full),
            pl.BlockSpec((_D, _D), full), pl.BlockSpec((_D, 1), full),
            pl.BlockSpec((_D, _D), full), pl.BlockSpec((_D, _D), full),
            pl.BlockSpec((_D, 1), full),
        ]
        wvals += [sch_fW1[k].T, sch_fb1[k].reshape(_D, 1),
                  sch_fW2[k].T, sch_fb2[k].reshape(_D, 1),
                  sch_W1[k].T, sch_W2[k].T, sch_b2[k].reshape(_D, 1)]
    rowb3 = lambda c: (c, 0, 0)
    return pl.pallas_call(
        _schnet_body,
        grid=(_NC // _CB,),
        in_specs=[pl.BlockSpec((1, 1, _NP), rowb3)] * 6 + [
            pl.BlockSpec((1, 1, _CA), rowb3),
            pl.BlockSpec((_D, 100), full),
        ] + wspecs,
        out_specs=pl.BlockSpec((1, _D, _CB), rowb3),
        out_shape=jax.ShapeDtypeStruct((_NC // _CB, _D, _CB), jnp.float32),
    )(*pairs, zrow, embt, *wvals)


def _final_body(sums_ref, cnt_ref, h2t_ref, mask_ref,
                w1at_ref, w1bt_ref, b1_ref, w2t_ref, b2_ref, out_ref):
    h1 = sums_ref[...] / jnp.maximum(cnt_ref[...], 1.0)
    h1t = h1.T
    g1 = jnp.dot(w1at_ref[...], h1t,
                 preferred_element_type=jnp.float32)
    mm = lax.broadcasted_iota(jnp.int32, (_NM, _NC), 0)
    cc = lax.broadcasted_iota(jnp.int32, (_NM, _NC), 1) // 10
    rmat = (mm == cc).astype(jnp.float32)
    g1r = jnp.dot(g1, rmat, preferred_element_type=jnp.float32)
    h2m = h2t_ref[...] * mask_ref[...]
    t = jnp.maximum(
        jnp.dot(w1bt_ref[...], h2m, preferred_element_type=jnp.float32)
        + g1r + b1_ref[...], 0.0)
    out_ref[...] = (jnp.dot(w2t_ref[...], t,
                            preferred_element_type=jnp.float32)
                    + b2_ref[...]) * mask_ref[...]


def _final(sums, cnt, h2t, maskr, w1at, w1bt, b1c, w2t, b2):
    return pl.pallas_call(
        _final_body,
        in_specs=[
            pl.BlockSpec(sums.shape, None), pl.BlockSpec(cnt.shape, None),
            pl.BlockSpec(h2t.shape, None), pl.BlockSpec(maskr.shape, None),
            pl.BlockSpec(w1at.shape, None), pl.BlockSpec(w1bt.shape, None),
            pl.BlockSpec(b1c.shape, None), pl.BlockSpec(w2t.shape, None),
            pl.BlockSpec(b2.shape, None),
        ],
        out_specs=pl.BlockSpec((1, _NC), None),
        out_shape=jax.ShapeDtypeStruct((1, _NC), jnp.float32),
    )(sums, cnt, h2t, maskr, w1at, w1bt, b1c, w2t, b2)


def kernel(x_solvent, edge_index_solvent, edge_attr_solvent, mol_attr_solvent,
           x_solvent_batch, z_solute, pos_solute, solute_confs_batch,
           solute_mask, max_confs,
           gnn_Wm, gnn_bm, gnn_Wu, gnn_bu, emb,
           sch_fW1, sch_fb1, sch_fW2, sch_fb2, sch_W1, sch_W2, sch_b2,
           ffn_W1, ffn_b1, ffn_W2, ffn_b2):
    src3 = edge_index_solvent[0].astype(jnp.int32).reshape(
        _NW, _NCH // _IG, _IG, _K)
    dst3 = edge_index_solvent[1].astype(jnp.int32).reshape(
        _NW, _NCH // _IG, _IG, _K)
    wme = jnp.stack([w[_D:, :] for w in gnn_Wm])
    wmx = [w[:_D, :] for w in gnn_Wm]
    wux = [w[:_D, :] for w in gnn_Wu]
    wua = [w[_D:, :] for w in gnn_Wu]
    bm2 = [b.reshape(1, _D) for b in gnn_bm]
    bu2 = [b.reshape(1, _D) for b in gnn_bu]
    bat3 = x_solvent_batch.astype(jnp.int32).reshape(10, 1, 1000)

    nstep = _NC // _CB

    def _pi(col):
        return jnp.broadcast_to(col[:, None], (_NC * _A, _A)).reshape(
            nstep, 1, _NP)

    def _pj(col):
        return jnp.broadcast_to(col.reshape(_NC, 1, _A),
                                (_NC, _A, _A)).reshape(nstep, 1, _NP)

    pairs = []
    for c in range(3):
        col = pos_solute[:, c]
        pairs += [_pi(col), _pj(col)]
    zrow = z_solute.astype(jnp.int32).reshape(nstep, 1, _CA)
    maskr = solute_mask.astype(jnp.float32).reshape(1, _NC)

    ep = _eproj(edge_attr_solvent, wme)
    x = x_solvent
    y = _lin(x, wmx[0], bm2[0])
    for l in range(_DEPTH):
        agg2 = _sc_agg(y, src3, dst3, ep[l])
        if l < _DEPTH - 1:
            x, y = _upd(x, agg2, wux[l], wua[l], bu2[l],
                        wmx[l + 1], bm2[l + 1])
        else:
            sums, cnt = _updh1(x, agg2, wux[l], wua[l], bu2[l], bat3)

    h2t3 = _schnet(pairs, zrow, emb.T,
                   sch_fW1, sch_fb1, sch_fW2, sch_fb2, sch_W1, sch_W2, sch_b2)
    h2t = jnp.swapaxes(h2t3, 0, 1).reshape(_D, _NC)

    out = _final(sums, cnt, h2t, maskr,
                 ffn_W1[:_D, :].T, ffn_W1[_D:, :].T,
                 ffn_b1.reshape(2 * _D, 1), ffn_W2.T, ffn_b2.reshape(1, 1))
    return out[0]


# speedup vs baseline: 2.6276x; 1.2602x over previous
import functools

import jax
import jax.numpy as jnp
from jax import lax
from jax.experimental import pallas as pl
from jax.experimental.pallas import tpu as pltpu
from jax.experimental.pallas import tpu_sc as plsc

_NN = 10000
_NE = 320000
_D = 128
_DE = 16
_NM = 128
_A = 16
_NC = 1280
_NG = 50
_CUT = 10.0
_DEPTH = 3
_NBLK = 2

_SC_C = 2
_SC_S = 16
_NW = _SC_C * _SC_S
_EPW = _NE // _NW
_K = 80
_NCH = _EPW // _K
_IG = 25


def _ssp(x):
    return jnp.maximum(x, 0.0) + jnp.log1p(jnp.exp(-jnp.abs(x))) - 0.6931471805599453


def _eproj_body(ea_ref, w_ref, out_ref):
    out_ref[0] = jnp.dot(ea_ref[...], w_ref[0],
                         preferred_element_type=jnp.float32)


def _eproj(edge_attr, w_all):
    eb = 2000
    return pl.pallas_call(
        _eproj_body,
        grid=(_DEPTH, _NE // eb),
        in_specs=[
            pl.BlockSpec((eb, _DE), lambda l, e: (e, 0)),
            pl.BlockSpec((1, _DE, _D), lambda l, e: (l, 0, 0)),
        ],
        out_specs=pl.BlockSpec((1, eb, _D), lambda l, e: (l, e, 0)),
        out_shape=jax.ShapeDtypeStruct((_DEPTH, _NE, _D), jnp.float32),
    )(edge_attr, w_all)


def _lin_body(x_ref, w_ref, b_ref, out_ref):
    out_ref[...] = jnp.dot(x_ref[...], w_ref[...],
                           preferred_element_type=jnp.float32) + b_ref[...]


def _lin(x, w, b):
    rb = 1000
    return pl.pallas_call(
        _lin_body,
        grid=(_NN // rb,),
        in_specs=[
            pl.BlockSpec((rb, _D), lambda r: (r, 0)),
            pl.BlockSpec((_D, _D), lambda r: (0, 0)),
            pl.BlockSpec((1, _D), lambda r: (0, 0)),
        ],
        out_specs=pl.BlockSpec((rb, _D), lambda r: (r, 0)),
        out_shape=jax.ShapeDtypeStruct((_NN, _D), jnp.float32),
    )(x, w, b)


def _sc_agg_body(y_hbm, src_hbm, dst_hbm, ep_hbm, out_hbm,
                 sidx, didx, buf,
                 se0, se1, se2, sg0, sg1, sg2, ss0, ss1, ss2,
                 aggs):
    cid = lax.axis_index("c")
    sid = lax.axis_index("s")
    wid = cid * _SC_S + sid
    sems_e = (se0, se1, se2)
    sems_g = (sg0, sg1, sg2)
    sems_s = (ss0, ss1, ss2)

    def _zr(r, _):
        for j in range(_D // 16):
            buf[0, r, pl.ds(16 * j, 16)] = jnp.zeros((16,), jnp.float32)
        return _
    lax.fori_loop(0, _K, _zr, None)
    off = sid * 640
    nz = jnp.where(sid == _SC_S - 1, 5, 8)

    def _zs(zc, _):
        pltpu.sync_copy(buf.at[0], aggs.at[pl.ds(off + 80 * zc, 80)])
        return _
    lax.fori_loop(0, nz, _zs, None)
    plsc.subcore_barrier()

    def _relu(b):
        def _row(r, ___):
            for j in range(_D // 16):
                s = pl.ds(16 * j, 16)
                buf[b, r, s] = jnp.maximum(buf[b, r, s], 0.0)
            return ___
        lax.fori_loop(0, _K, _row, None)

    def _group(g, _):
        pltpu.sync_copy(src_hbm.at[wid, g], sidx)
        pltpu.sync_copy(dst_hbm.at[wid, g], didx)
        dsc = {}
        for t in range(_IG + 2):
            if t < _IG:
                b = t % 3
                if t >= 3:
                    dsc['s', t - 3].wait()
                base = (wid * _EPW // _K + g * _IG + t) * _K
                dsc['e', t] = pltpu.async_copy(
                    ep_hbm.at[pl.ds(base, _K)], buf.at[b], sems_e[b])
            if t >= 1 and t - 1 < _IG:
                b = (t - 1) % 3
                dsc['e', t - 1].wait()
                dsc['g', t - 1] = pltpu.async_copy(
                    y_hbm.at[sidx.at[t - 1]], buf.at[b], sems_g[b], add=True)
            if t >= 2 and t - 2 < _IG:
                b = (t - 2) % 3
                dsc['g', t - 2].wait()
                _relu(b)
                dsc['s', t - 2] = pltpu.async_copy(
                    buf.at[b], aggs.at[didx.at[t - 2]], sems_s[b], add=True)
        for t in (_IG - 3, _IG - 2, _IG - 1):
            dsc['s', t].wait()
        return _
    lax.fori_loop(0, _NCH // _IG, _group, None)
    plsc.subcore_barrier()

    def _out(zc, _):
        o = off + 80 * zc
        pltpu.sync_copy(aggs.at[pl.ds(o, 80)], out_hbm.at[cid, pl.ds(o, 80)])
        return _
    lax.fori_loop(0, nz, _out, None)


def _sc_agg(y, src3, dst3, ep_l):
    mesh = plsc.VectorSubcoreMesh(core_axis_name="c", subcore_axis_name="s")
    f = pl.kernel(
        _sc_agg_body,
        out_type=jax.ShapeDtypeStruct((_SC_C, _NN, _D), jnp.float32),
        mesh=mesh,
        scratch_types=[
            pltpu.VMEM((_IG, _K), jnp.int32),
            pltpu.VMEM((_IG, _K), jnp.int32),
            pltpu.VMEM((3, _K, _D), jnp.float32),
        ] + [pltpu.SemaphoreType.DMA] * 9 + [
            pltpu.VMEM_SHARED((_NN, _D), jnp.float32),
        ],
    )
    return f(y, src3, dst3, ep_l)


def _upd_body(x_ref, a0_ref, a1_ref, wux_ref, wua_ref, bu_ref,
              wmx_ref, bm_ref, xn_ref, yn_ref):
    agg = a0_ref[0] + a1_ref[0]
    xn = jnp.maximum(
        jnp.dot(x_ref[...], wux_ref[...], preferred_element_type=jnp.float32)
        + jnp.dot(agg, wua_ref[...], preferred_element_type=jnp.float32)
        + bu_ref[...], 0.0)
    xn_ref[...] = xn
    yn_ref[...] = jnp.dot(xn, wmx_ref[...],
                          preferred_element_type=jnp.float32) + bm_ref[...]


def _upd(x, agg2, wux, wua, bu, wmx, bm):
    rb = 1000
    full = lambda r: (0, 0)
    return pl.pallas_call(
        _upd_body,
        grid=(_NN // rb,),
        in_specs=[
            pl.BlockSpec((rb, _D), lambda r: (r, 0)),
            pl.BlockSpec((1, rb, _D), lambda r: (0, r, 0)),
            pl.BlockSpec((1, rb, _D), lambda r: (1, r, 0)),
            pl.BlockSpec((_D, _D), full), pl.BlockSpec((_D, _D), full),
            pl.BlockSpec((1, _D), full),
            pl.BlockSpec((_D, _D), full), pl.BlockSpec((1, _D), full),
        ],
        out_specs=[
            pl.BlockSpec((rb, _D), lambda r: (r, 0)),
            pl.BlockSpec((rb, _D), lambda r: (r, 0)),
        ],
        out_shape=[
            jax.ShapeDtypeStruct((_NN, _D), jnp.float32),
            jax.ShapeDtypeStruct((_NN, _D), jnp.float32),
        ],
    )(x, agg2, agg2, wux, wua, bu, wmx, bm)


def _upd_body_a0(a0_ref, xn_ref):
    xn_ref[...] = a0_ref[...]


def _updh1_body(x_ref, a0_ref, a1_ref, wux_ref, wua_ref, bu_ref, bat_ref,
                sums_ref, cnt_ref):
    @pl.when(pl.program_id(0) == 0)
    def _init():
        sums_ref[...] = jnp.zeros_like(sums_ref)
        cnt_ref[...] = jnp.zeros_like(cnt_ref)

    agg = a0_ref[0] + a1_ref[0]
    xn = jnp.maximum(
        jnp.dot(x_ref[...], wux_ref[...], preferred_element_type=jnp.float32)
        + jnp.dot(agg, wua_ref[...], preferred_element_type=jnp.float32)
        + bu_ref[...], 0.0)
    b = bat_ref[0, 0, :]
    rows = lax.broadcasted_iota(jnp.int32, (_NM, b.shape[0]), 0)
    oh = (rows == b[None, :]).astype(jnp.float32)
    sums_ref[...] += jnp.dot(oh, xn, preferred_element_type=jnp.float32,
                             precision=lax.Precision.HIGHEST)
    cnt_ref[...] += jnp.sum(oh, axis=1, keepdims=True)


def _updh1(x, agg2, wux, wua, bu, bat3):
    rb = 1000
    full = lambda r: (0, 0)
    return pl.pallas_call(
        _updh1_body,
        grid=(_NN // rb,),
        in_specs=[
            pl.BlockSpec((rb, _D), lambda r: (r, 0)),
            pl.BlockSpec((1, rb, _D), lambda r: (0, r, 0)),
            pl.BlockSpec((1, rb, _D), lambda r: (1, r, 0)),
            pl.BlockSpec((_D, _D), full), pl.BlockSpec((_D, _D), full),
            pl.BlockSpec((1, _D), full),
            pl.BlockSpec((1, 1, rb), lambda r: (r, 0, 0)),
        ],
        out_specs=[
            pl.BlockSpec((_NM, _D), full),
            pl.BlockSpec((_NM, 1), full),
        ],
        out_shape=[
            jax.ShapeDtypeStruct((_NM, _D), jnp.float32),
            jax.ShapeDtypeStruct((_NM, 1), jnp.float32),
        ],
    )(x, agg2, agg2, wux, wua, bu, bat3)


_CB = 8
_CA = _CB * _A


_NP = _CA * _A


def _schnet_body(pxi_ref, pxj_ref, pyi_ref, pyj_ref, pzi_ref, pzj_ref,
                 z_ref, emb_ref,
                 fw1a_ref, fb1a_ref, fw2a_ref, fb2a_ref,
                 w1a_ref, w2a_ref, b2a_ref,
                 fw1b_ref, fb1b_ref, fw2b_ref, fb2b_ref,
                 w1b_ref, w2b_ref, b2b_ref,
                 h2_ref):
    dx = pxi_ref[0] - pxj_ref[0]
    dy = pyi_ref[0] - pyj_ref[0]
    dz = pzi_ref[0] - pzj_ref[0]
    d = jnp.sqrt(dx * dx + dy * dy + dz * dz + 1e-12)

    mu = lax.broadcasted_iota(jnp.int32, (_NG, 1), 0).astype(
        jnp.float32) * (_CUT / (_NG - 1))
    rbft = jnp.exp(-10.0 * (d - mu) ** 2)

    p = lax.broadcasted_iota(jnp.int32, (1, _NP), 1)
    gate_row = (0.5 * (jnp.cos(jnp.pi * d / _CUT) + 1.0) * (d < _CUT)
                * ((p // _A) % _A != p % _A))

    rbf2 = rbft.T
    gate = gate_row.T

    zi = z_ref[0].T
    ks = lax.broadcasted_iota(jnp.int32, (_CA, 100), 1)
    oh = (zi == ks).astype(jnp.float32)
    h = jnp.dot(oh, emb_ref[...], preferred_element_type=jnp.float32,
                precision=lax.Precision.HIGHEST)

    blocks = [
        (fw1a_ref, fb1a_ref, fw2a_ref, fb2a_ref, w1a_ref, w2a_ref, b2a_ref),
        (fw1b_ref, fb1b_ref, fw2b_ref, fb2b_ref, w1b_ref, w2b_ref, b2b_ref),
    ]
    for fw1, fb1, fw2, fb2, w1, w2, b2 in blocks:
        t1 = _ssp(jnp.dot(rbf2, fw1[...],
                          preferred_element_type=jnp.float32) + fb1[...])
        filt = (jnp.dot(t1, fw2[...],
                        preferred_element_type=jnp.float32) + fb2[...]) * gate
        v = jnp.dot(h, w1[...], preferred_element_type=jnp.float32)
        f4 = filt.reshape(_CB, _A, _A, _D)
        v4 = v.reshape(_CB, 1, _A, _D)
        m = jnp.sum(f4 * v4, axis=2).reshape(_CA, _D)
        h = h + jnp.dot(_ssp(m), w2[...],
                        preferred_element_type=jnp.float32) + b2[...]

    h2_ref[...] = jnp.sum(h.reshape(_CB, _A, _D), axis=1)


def _schnet(pairs, zrow, emb,
            sch_fW1, sch_fb1, sch_fW2, sch_fb2, sch_W1, sch_W2, sch_b2):
    full = lambda c: (0, 0)
    wspecs = []
    wvals = []
    for k in range(_NBLK):
        wspecs += [
            pl.BlockSpec((_NG, _D), full), pl.BlockSpec((1, _D), full),
            pl.BlockSpec((_D, _D), full), pl.BlockSpec((1, _D), full),
            pl.BlockSpec((_D, _D), full), pl.BlockSpec((_D, _D), full),
            pl.BlockSpec((1, _D), full),
        ]
        wvals += [sch_fW1[k], sch_fb1[k].reshape(1, _D),
                  sch_fW2[k], sch_fb2[k].reshape(1, _D),
                  sch_W1[k], sch_W2[k], sch_b2[k].reshape(1, _D)]
    rowb3 = lambda c: (c, 0, 0)
    return pl.pallas_call(
        _schnet_body,
        grid=(_NC // _CB,),
        in_specs=[pl.BlockSpec((1, 1, _NP), rowb3)] * 6 + [
            pl.BlockSpec((1, 1, _CA), rowb3),
            pl.BlockSpec((100, _D), full),
        ] + wspecs,
        out_specs=pl.BlockSpec((_CB, _D), lambda c: (c, 0)),
        out_shape=jax.ShapeDtypeStruct((_NC, _D), jnp.float32),
    )(*pairs, zrow, emb, *wvals)


def _final_body(sums_ref, cnt_ref, h2t_ref, mask_ref,
                w1at_ref, w1bt_ref, b1_ref, w2t_ref, b2_ref, out_ref):
    h1 = sums_ref[...] / jnp.maximum(cnt_ref[...], 1.0)
    h1t = h1.T
    g1 = jnp.dot(w1at_ref[...], h1t,
                 preferred_element_type=jnp.float32)
    mm = lax.broadcasted_iota(jnp.int32, (_NM, _NC), 0)
    cc = lax.broadcasted_iota(jnp.int32, (_NM, _NC), 1) // 10
    rmat = (mm == cc).astype(jnp.float32)
    g1r = jnp.dot(g1, rmat, preferred_element_type=jnp.float32,
                  precision=lax.Precision.HIGHEST)
    h2m = h2t_ref[...] * mask_ref[...]
    t = jnp.maximum(
        jnp.dot(w1bt_ref[...], h2m, preferred_element_type=jnp.float32)
        + g1r + b1_ref[...], 0.0)
    out_ref[...] = (jnp.dot(w2t_ref[...], t,
                            preferred_element_type=jnp.float32)
                    + b2_ref[...]) * mask_ref[...]


def _final(sums, cnt, h2t, maskr, w1at, w1bt, b1c, w2t, b2):
    return pl.pallas_call(
        _final_body,
        in_specs=[
            pl.BlockSpec(sums.shape, None), pl.BlockSpec(cnt.shape, None),
            pl.BlockSpec(h2t.shape, None), pl.BlockSpec(maskr.shape, None),
            pl.BlockSpec(w1at.shape, None), pl.BlockSpec(w1bt.shape, None),
            pl.BlockSpec(b1c.shape, None), pl.BlockSpec(w2t.shape, None),
            pl.BlockSpec(b2.shape, None),
        ],
        out_specs=pl.BlockSpec((1, _NC), None),
        out_shape=jax.ShapeDtypeStruct((1, _NC), jnp.float32),
    )(sums, cnt, h2t, maskr, w1at, w1bt, b1c, w2t, b2)


def kernel(x_solvent, edge_index_solvent, edge_attr_solvent, mol_attr_solvent,
           x_solvent_batch, z_solute, pos_solute, solute_confs_batch,
           solute_mask, max_confs,
           gnn_Wm, gnn_bm, gnn_Wu, gnn_bu, emb,
           sch_fW1, sch_fb1, sch_fW2, sch_fb2, sch_W1, sch_W2, sch_b2,
           ffn_W1, ffn_b1, ffn_W2, ffn_b2):
    src3 = edge_index_solvent[0].astype(jnp.int32).reshape(
        _NW, _NCH // _IG, _IG, _K)
    dst3 = edge_index_solvent[1].astype(jnp.int32).reshape(
        _NW, _NCH // _IG, _IG, _K)
    wme = jnp.stack([w[_D:, :] for w in gnn_Wm])
    wmx = [w[:_D, :] for w in gnn_Wm]
    wux = [w[:_D, :] for w in gnn_Wu]
    wua = [w[_D:, :] for w in gnn_Wu]
    bm2 = [b.reshape(1, _D) for b in gnn_bm]
    bu2 = [b.reshape(1, _D) for b in gnn_bu]
    bat3 = x_solvent_batch.astype(jnp.int32).reshape(10, 1, 1000)

    nstep = _NC // _CB

    def _pi(col):
        return jnp.broadcast_to(col[:, None], (_NC * _A, _A)).reshape(
            nstep, 1, _NP)

    def _pj(col):
        return jnp.broadcast_to(col.reshape(_NC, 1, _A),
                                (_NC, _A, _A)).reshape(nstep, 1, _NP)

    pairs = []
    for c in range(3):
        col = pos_solute[:, c]
        pairs += [_pi(col), _pj(col)]
    zrow = z_solute.astype(jnp.int32).reshape(nstep, 1, _CA)
    maskr = solute_mask.astype(jnp.float32).reshape(1, _NC)

    ep = _eproj(edge_attr_solvent, wme)
    x = x_solvent
    y = _lin(x, wmx[0], bm2[0])
    for l in range(_DEPTH):
        agg2 = _sc_agg(y, src3, dst3, ep[l])
        if l < _DEPTH - 1:
            x, y = _upd(x, agg2, wux[l], wua[l], bu2[l],
                        wmx[l + 1], bm2[l + 1])
        else:
            sums, cnt = _updh1(x, agg2, wux[l], wua[l], bu2[l], bat3)

    h2 = _schnet(pairs, zrow, emb,
                 sch_fW1, sch_fb1, sch_fW2, sch_fb2, sch_W1, sch_W2, sch_b2)
    h2t = h2.T

    out = _final(sums, cnt, h2t, maskr,
                 ffn_W1[:_D, :].T, ffn_W1[_D:, :].T,
                 ffn_b1.reshape(2 * _D, 1), ffn_W2.T, ffn_b2.reshape(1, 1))
    return out[0]


# speedup vs baseline: 2.6291x; 1.0006x over previous
import functools

import jax
import jax.numpy as jnp
from jax import lax
from jax.experimental import pallas as pl
from jax.experimental.pallas import tpu as pltpu
from jax.experimental.pallas import tpu_sc as plsc

_NN = 10000
_NE = 320000
_D = 128
_DE = 16
_NM = 128
_A = 16
_NC = 1280
_NG = 50
_CUT = 10.0
_DEPTH = 3
_NBLK = 2

_SC_C = 2
_SC_S = 16
_NW = _SC_C * _SC_S
_EPW = _NE // _NW
_K = 80
_NCH = _EPW // _K
_IG = 25


def _ssp(x):
    return jnp.maximum(x, 0.0) + jnp.log1p(jnp.exp(-jnp.abs(x))) - 0.6931471805599453


def _eproj_body(ea_ref, w_ref, out_ref):
    out_ref[0] = jnp.dot(ea_ref[...], w_ref[0],
                         preferred_element_type=jnp.float32)


def _eproj(edge_attr, w_all):
    eb = 2000
    return pl.pallas_call(
        _eproj_body,
        grid=(_DEPTH, _NE // eb),
        in_specs=[
            pl.BlockSpec((eb, _DE), lambda l, e: (e, 0)),
            pl.BlockSpec((1, _DE, _D), lambda l, e: (l, 0, 0)),
        ],
        out_specs=pl.BlockSpec((1, eb, _D), lambda l, e: (l, e, 0)),
        out_shape=jax.ShapeDtypeStruct((_DEPTH, _NE, _D), jnp.float32),
    )(edge_attr, w_all)


def _lin_body(x_ref, w_ref, b_ref, out_ref):
    out_ref[...] = jnp.dot(x_ref[...], w_ref[...],
                           preferred_element_type=jnp.float32) + b_ref[...]


def _lin(x, w, b):
    rb = 1000
    return pl.pallas_call(
        _lin_body,
        grid=(_NN // rb,),
        in_specs=[
            pl.BlockSpec((rb, _D), lambda r: (r, 0)),
            pl.BlockSpec((_D, _D), lambda r: (0, 0)),
            pl.BlockSpec((1, _D), lambda r: (0, 0)),
        ],
        out_specs=pl.BlockSpec((rb, _D), lambda r: (r, 0)),
        out_shape=jax.ShapeDtypeStruct((_NN, _D), jnp.float32),
    )(x, w, b)


def _sc_agg_body(y_hbm, src_hbm, dst_hbm, ep_hbm, out_hbm,
                 sidx, didx, buf,
                 se0, se1, se2, sg0, sg1, sg2,
                 aggs):
    cid = lax.axis_index("c")
    sid = lax.axis_index("s")
    wid = cid * _SC_S + sid
    sems_e = (se0, se1, se2)
    sems_g = (sg0, sg1, sg2)

    def _zr(r, _):
        for j in range(_D // 16):
            buf[0, r, pl.ds(16 * j, 16)] = jnp.zeros((16,), jnp.float32)
        return _
    lax.fori_loop(0, _K, _zr, None)
    off = sid * 640
    nz = jnp.where(sid == _SC_S - 1, 5, 8)

    def _zs(zc, _):
        pltpu.sync_copy(buf.at[0], aggs.at[pl.ds(off + 80 * zc, 80)])
        return _
    lax.fori_loop(0, nz, _zs, None)
    plsc.subcore_barrier()

    def _relu(b):
        def _row(r, ___):
            for j in range(_D // 16):
                s = pl.ds(16 * j, 16)
                buf[b, r, s] = jnp.maximum(buf[b, r, s], 0.0)
            return ___
        lax.fori_loop(0, _K, _row, None)

    def _group(g, _):
        pltpu.sync_copy(src_hbm.at[wid, g], sidx)
        pltpu.sync_copy(dst_hbm.at[wid, g], didx)
        dsc = {}
        for t in range(_IG + 2):
            if t < _IG:
                b = t % 3
                base = (wid * _EPW // _K + g * _IG + t) * _K
                dsc['e', t] = pltpu.async_copy(
                    ep_hbm.at[pl.ds(base, _K)], buf.at[b], sems_e[b])
            if t >= 1 and t - 1 < _IG:
                b = (t - 1) % 3
                dsc['e', t - 1].wait()
                dsc['g', t - 1] = pltpu.async_copy(
                    y_hbm.at[sidx.at[t - 1]], buf.at[b], sems_g[b], add=True)
            if t >= 2 and t - 2 < _IG:
                b = (t - 2) % 3
                dsc['g', t - 2].wait()
                _relu(b)
                pltpu.sync_copy(buf.at[b], aggs.at[didx.at[t - 2]], add=True)
        return _
    lax.fori_loop(0, _NCH // _IG, _group, None)
    plsc.subcore_barrier()

    def _out(zc, _):
        o = off + 80 * zc
        pltpu.sync_copy(aggs.at[pl.ds(o, 80)], out_hbm.at[cid, pl.ds(o, 80)])
        return _
    lax.fori_loop(0, nz, _out, None)


def _sc_agg(y, src3, dst3, ep_l):
    mesh = plsc.VectorSubcoreMesh(core_axis_name="c", subcore_axis_name="s")
    f = pl.kernel(
        _sc_agg_body,
        out_type=jax.ShapeDtypeStruct((_SC_C, _NN, _D), jnp.float32),
        mesh=mesh,
        scratch_types=[
            pltpu.VMEM((_IG, _K), jnp.int32),
            pltpu.VMEM((_IG, _K), jnp.int32),
            pltpu.VMEM((3, _K, _D), jnp.float32),
        ] + [pltpu.SemaphoreType.DMA] * 6 + [
            pltpu.VMEM_SHARED((_NN, _D), jnp.float32),
        ],
    )
    return f(y, src3, dst3, ep_l)


def _upd_body(x_ref, a0_ref, a1_ref, wux_ref, wua_ref, bu_ref,
              wmx_ref, bm_ref, xn_ref, yn_ref):
    agg = a0_ref[0] + a1_ref[0]
    xn = jnp.maximum(
        jnp.dot(x_ref[...], wux_ref[...], preferred_element_type=jnp.float32)
        + jnp.dot(agg, wua_ref[...], preferred_element_type=jnp.float32)
        + bu_ref[...], 0.0)
    xn_ref[...] = xn
    yn_ref[...] = jnp.dot(xn, wmx_ref[...],
                          preferred_element_type=jnp.float32) + bm_ref[...]


def _upd(x, agg2, wux, wua, bu, wmx, bm):
    rb = 1000
    full = lambda r: (0, 0)
    return pl.pallas_call(
        _upd_body,
        grid=(_NN // rb,),
        in_specs=[
            pl.BlockSpec((rb, _D), lambda r: (r, 0)),
            pl.BlockSpec((1, rb, _D), lambda r: (0, r, 0)),
            pl.BlockSpec((1, rb, _D), lambda r: (1, r, 0)),
            pl.BlockSpec((_D, _D), full), pl.BlockSpec((_D, _D), full),
            pl.BlockSpec((1, _D), full),
            pl.BlockSpec((_D, _D), full), pl.BlockSpec((1, _D), full),
        ],
        out_specs=[
            pl.BlockSpec((rb, _D), lambda r: (r, 0)),
            pl.BlockSpec((rb, _D), lambda r: (r, 0)),
        ],
        out_shape=[
            jax.ShapeDtypeStruct((_NN, _D), jnp.float32),
            jax.ShapeDtypeStruct((_NN, _D), jnp.float32),
        ],
    )(x, agg2, agg2, wux, wua, bu, wmx, bm)


def _upd_body_a0(a0_ref, xn_ref):
    xn_ref[...] = a0_ref[...]


def _updh1_body(x_ref, a0_ref, a1_ref, wux_ref, wua_ref, bu_ref, bat_ref,
                sums_ref, cnt_ref):
    @pl.when(pl.program_id(0) == 0)
    def _init():
        sums_ref[...] = jnp.zeros_like(sums_ref)
        cnt_ref[...] = jnp.zeros_like(cnt_ref)

    agg = a0_ref[0] + a1_ref[0]
    xn = jnp.maximum(
        jnp.dot(x_ref[...], wux_ref[...], preferred_element_type=jnp.float32)
        + jnp.dot(agg, wua_ref[...], preferred_element_type=jnp.float32)
        + bu_ref[...], 0.0)
    b = bat_ref[0, 0, :]
    rows = lax.broadcasted_iota(jnp.int32, (_NM, b.shape[0]), 0)
    oh = (rows == b[None, :]).astype(jnp.float32)
    sums_ref[...] += jnp.dot(oh, xn, preferred_element_type=jnp.float32,
                             precision=lax.Precision.HIGHEST)
    cnt_ref[...] += jnp.sum(oh, axis=1, keepdims=True)


def _updh1(x, agg2, wux, wua, bu, bat3):
    rb = 1000
    full = lambda r: (0, 0)
    return pl.pallas_call(
        _updh1_body,
        grid=(_NN // rb,),
        in_specs=[
            pl.BlockSpec((rb, _D), lambda r: (r, 0)),
            pl.BlockSpec((1, rb, _D), lambda r: (0, r, 0)),
            pl.BlockSpec((1, rb, _D), lambda r: (1, r, 0)),
            pl.BlockSpec((_D, _D), full), pl.BlockSpec((_D, _D), full),
            pl.BlockSpec((1, _D), full),
            pl.BlockSpec((1, 1, rb), lambda r: (r, 0, 0)),
        ],
        out_specs=[
            pl.BlockSpec((_NM, _D), full),
            pl.BlockSpec((_NM, 1), full),
        ],
        out_shape=[
            jax.ShapeDtypeStruct((_NM, _D), jnp.float32),
            jax.ShapeDtypeStruct((_NM, 1), jnp.float32),
        ],
    )(x, agg2, agg2, wux, wua, bu, bat3)


_CB = 8
_CA = _CB * _A


_NP = _CA * _A


def _schnet_body(pxi_ref, pxj_ref, pyi_ref, pyj_ref, pzi_ref, pzj_ref,
                 z_ref, emb_ref,
                 fw1a_ref, fb1a_ref, fw2a_ref, fb2a_ref,
                 w1a_ref, w2a_ref, b2a_ref,
                 fw1b_ref, fb1b_ref, fw2b_ref, fb2b_ref,
                 w1b_ref, w2b_ref, b2b_ref,
                 h2_ref):
    dx = pxi_ref[0] - pxj_ref[0]
    dy = pyi_ref[0] - pyj_ref[0]
    dz = pzi_ref[0] - pzj_ref[0]
    d = jnp.sqrt(dx * dx + dy * dy + dz * dz + 1e-12)

    mu = lax.broadcasted_iota(jnp.int32, (_NG, 1), 0).astype(
        jnp.float32) * (_CUT / (_NG - 1))
    rbft = jnp.exp(-10.0 * (d - mu) ** 2)

    p = lax.broadcasted_iota(jnp.int32, (1, _NP), 1)
    gate_row = (0.5 * (jnp.cos(jnp.pi * d / _CUT) + 1.0) * (d < _CUT)
                * ((p // _A) % _A != p % _A))

    rbf2 = rbft.T
    gate = gate_row.T

    zi = z_ref[0].T
    ks = lax.broadcasted_iota(jnp.int32, (_CA, 100), 1)
    oh = (zi == ks).astype(jnp.float32)
    h = jnp.dot(oh, emb_ref[...], preferred_element_type=jnp.float32,
                precision=lax.Precision.HIGHEST)

    blocks = [
        (fw1a_ref, fb1a_ref, fw2a_ref, fb2a_ref, w1a_ref, w2a_ref, b2a_ref),
        (fw1b_ref, fb1b_ref, fw2b_ref, fb2b_ref, w1b_ref, w2b_ref, b2b_ref),
    ]
    for fw1, fb1, fw2, fb2, w1, w2, b2 in blocks:
        t1 = _ssp(jnp.dot(rbf2, fw1[...],
                          preferred_element_type=jnp.float32) + fb1[...])
        filt = (jnp.dot(t1, fw2[...],
                        preferred_element_type=jnp.float32) + fb2[...]) * gate
        v = jnp.dot(h, w1[...], preferred_element_type=jnp.float32)
        f4 = filt.reshape(_CB, _A, _A, _D)
        v4 = v.reshape(_CB, 1, _A, _D)
        m = jnp.sum(f4 * v4, axis=2).reshape(_CA, _D)
        h = h + jnp.dot(_ssp(m), w2[...],
                        preferred_element_type=jnp.float32) + b2[...]

    h2_ref[...] = jnp.sum(h.reshape(_CB, _A, _D), axis=1)


def _schnet(pairs, zrow, emb,
            sch_fW1, sch_fb1, sch_fW2, sch_fb2, sch_W1, sch_W2, sch_b2):
    full = lambda c: (0, 0)
    wspecs = []
    wvals = []
    for k in range(_NBLK):
        wspecs += [
            pl.BlockSpec((_NG, _D), full), pl.BlockSpec((1, _D), full),
            pl.BlockSpec((_D, _D), full), pl.BlockSpec((1, _D), full),
            pl.BlockSpec((_D, _D), full), pl.BlockSpec((_D, _D), full),
            pl.BlockSpec((1, _D), full),
        ]
        wvals += [sch_fW1[k], sch_fb1[k].reshape(1, _D),
                  sch_fW2[k], sch_fb2[k].reshape(1, _D),
                  sch_W1[k], sch_W2[k], sch_b2[k].reshape(1, _D)]
    rowb3 = lambda c: (c, 0, 0)
    return pl.pallas_call(
        _schnet_body,
        grid=(_NC // _CB,),
        in_specs=[pl.BlockSpec((1, 1, _NP), rowb3)] * 6 + [
            pl.BlockSpec((1, 1, _CA), rowb3),
            pl.BlockSpec((100, _D), full),
        ] + wspecs,
        out_specs=pl.BlockSpec((_CB, _D), lambda c: (c, 0)),
        out_shape=jax.ShapeDtypeStruct((_NC, _D), jnp.float32),
    )(*pairs, zrow, emb, *wvals)


def _final_body(sums_ref, cnt_ref, h2t_ref, mask_ref,
                w1at_ref, w1bt_ref, b1_ref, w2t_ref, b2_ref, out_ref):
    h1 = sums_ref[...] / jnp.maximum(cnt_ref[...], 1.0)
    h1t = h1.T
    g1 = jnp.dot(w1at_ref[...], h1t,
                 preferred_element_type=jnp.float32)
    mm = lax.broadcasted_iota(jnp.int32, (_NM, _NC), 0)
    cc = lax.broadcasted_iota(jnp.int32, (_NM, _NC), 1) // 10
    rmat = (mm == cc).astype(jnp.float32)
    g1r = jnp.dot(g1, rmat, preferred_element_type=jnp.float32,
                  precision=lax.Precision.HIGHEST)
    h2m = h2t_ref[...] * mask_ref[...]
    t = jnp.maximum(
        jnp.dot(w1bt_ref[...], h2m, preferred_element_type=jnp.float32)
        + g1r + b1_ref[...], 0.0)
    out_ref[...] = (jnp.dot(w2t_ref[...], t,
                            preferred_element_type=jnp.float32)
                    + b2_ref[...]) * mask_ref[...]


def _final(sums, cnt, h2t, maskr, w1at, w1bt, b1c, w2t, b2):
    return pl.pallas_call(
        _final_body,
        in_specs=[
            pl.BlockSpec(sums.shape, None), pl.BlockSpec(cnt.shape, None),
            pl.BlockSpec(h2t.shape, None), pl.BlockSpec(maskr.shape, None),
            pl.BlockSpec(w1at.shape, None), pl.BlockSpec(w1bt.shape, None),
            pl.BlockSpec(b1c.shape, None), pl.BlockSpec(w2t.shape, None),
            pl.BlockSpec(b2.shape, None),
        ],
        out_specs=pl.BlockSpec((1, _NC), None),
        out_shape=jax.ShapeDtypeStruct((1, _NC), jnp.float32),
    )(sums, cnt, h2t, maskr, w1at, w1bt, b1c, w2t, b2)


def kernel(x_solvent, edge_index_solvent, edge_attr_solvent, mol_attr_solvent,
           x_solvent_batch, z_solute, pos_solute, solute_confs_batch,
           solute_mask, max_confs,
           gnn_Wm, gnn_bm, gnn_Wu, gnn_bu, emb,
           sch_fW1, sch_fb1, sch_fW2, sch_fb2, sch_W1, sch_W2, sch_b2,
           ffn_W1, ffn_b1, ffn_W2, ffn_b2):
    src3 = edge_index_solvent[0].astype(jnp.int32).reshape(
        _NW, _NCH // _IG, _IG, _K)
    dst3 = edge_index_solvent[1].astype(jnp.int32).reshape(
        _NW, _NCH // _IG, _IG, _K)
    wme = jnp.stack([w[_D:, :] for w in gnn_Wm])
    wmx = [w[:_D, :] for w in gnn_Wm]
    wux = [w[:_D, :] for w in gnn_Wu]
    wua = [w[_D:, :] for w in gnn_Wu]
    bm2 = [b.reshape(1, _D) for b in gnn_bm]
    bu2 = [b.reshape(1, _D) for b in gnn_bu]
    bat3 = x_solvent_batch.astype(jnp.int32).reshape(10, 1, 1000)

    nstep = _NC // _CB

    def _pi(col):
        return jnp.broadcast_to(col[:, None], (_NC * _A, _A)).reshape(
            nstep, 1, _NP)

    def _pj(col):
        return jnp.broadcast_to(col.reshape(_NC, 1, _A),
                                (_NC, _A, _A)).reshape(nstep, 1, _NP)

    pairs = []
    for c in range(3):
        col = pos_solute[:, c]
        pairs += [_pi(col), _pj(col)]
    zrow = z_solute.astype(jnp.int32).reshape(nstep, 1, _CA)
    maskr = solute_mask.astype(jnp.float32).reshape(1, _NC)

    ep = _eproj(edge_attr_solvent, wme)
    x = x_solvent
    y = _lin(x, wmx[0], bm2[0])
    for l in range(_DEPTH):
        agg2 = _sc_agg(y, src3, dst3, ep[l])
        if l < _DEPTH - 1:
            x, y = _upd(x, agg2, wux[l], wua[l], bu2[l],
                        wmx[l + 1], bm2[l + 1])
        else:
            sums, cnt = _updh1(x, agg2, wux[l], wua[l], bu2[l], bat3)

    h2 = _schnet(pairs, zrow, emb,
                 sch_fW1, sch_fb1, sch_fW2, sch_fb2, sch_W1, sch_W2, sch_b2)
    h2t = h2.T

    out = _final(sums, cnt, h2t, maskr,
                 ffn_W1[:_D, :].T, ffn_W1[_D:, :].T,
                 ffn_b1.reshape(2 * _D, 1), ffn_W2.T, ffn_b2.reshape(1, 1))
    return out[0]
